# Initial kernel scaffold; baseline (speedup 1.0000x reference)
import jax
import jax.numpy as jnp
from jax.experimental import pallas as pl


def kernel(hidden_states, router_weight, merged_gate_up_proj, merged_down_proj, shared_gate_up_w, shared_down_w, shared_gate_w):
    raise NotImplementedError("write your pallas kernel here")


# speedup vs baseline: 2.3290x; 2.3290x over previous
import functools

import jax
import jax.numpy as jnp
from jax.experimental import pallas as pl
from jax.experimental.pallas import tpu as pltpu

E = 8
K = 2
D = 1024
F = 1408
FS = 5632
N = 2048

TOK_TILE = 512
FS_CHUNK = 1408


def _router_body(x_ref, wr_ref, c_ref):
    x = x_ref[...]
    wr = wr_ref[...]
    logits = jax.lax.dot_general(x, wr, (((1,), (1,)), ((), ())),
                                 preferred_element_type=jnp.float32)
    probs = jax.nn.softmax(logits, axis=-1)
    iota = jax.lax.broadcasted_iota(jnp.int32, probs.shape, 1)
    m1 = jnp.max(probs, axis=-1, keepdims=True)
    a1 = jnp.min(jnp.where(probs == m1, iota, E), axis=-1, keepdims=True)
    masked = jnp.where(iota == a1, -jnp.inf, probs)
    m2 = jnp.max(masked, axis=-1, keepdims=True)
    a2 = jnp.min(jnp.where(masked == m2, iota, E), axis=-1, keepdims=True)
    c = jnp.where(iota == a1, m1, 0.0) + jnp.where(iota == a2, m2, 0.0)
    c_ref[...] = c


def _expert_body(x_ref, c_ref, wgu_ref, wd_ref, out_ref):
    e = pl.program_id(1)
    x = x_ref[...]
    merged = jnp.dot(x, wgu_ref[0], preferred_element_type=jnp.float32)
    gate = merged[:, :F]
    up = merged[:, F:]
    h = jax.nn.silu(gate) * up
    o = jnp.dot(h, wd_ref[0], preferred_element_type=jnp.float32)
    iota = jax.lax.broadcasted_iota(jnp.int32, (TOK_TILE, E), 1)
    ce = jnp.sum(jnp.where(iota == e, c_ref[...], 0.0), axis=1, keepdims=True)

    @pl.when(e == 0)
    def _():
        out_ref[...] = jnp.zeros_like(out_ref)

    out_ref[...] += ce * o


def _shared_body(x_ref, wg_ref, wu_ref, wd_ref, eo_ref, gw_ref, out_ref,
                 acc_ref):
    j = pl.program_id(1)
    nj = pl.num_programs(1)
    x = x_ref[...]
    g = jax.lax.dot_general(x, wg_ref[...], (((1,), (1,)), ((), ())),
                            preferred_element_type=jnp.float32)
    u = jax.lax.dot_general(x, wu_ref[...], (((1,), (1,)), ((), ())),
                            preferred_element_type=jnp.float32)
    h = jax.nn.silu(g) * u
    o = jax.lax.dot_general(h, wd_ref[...], (((1,), (1,)), ((), ())),
                            preferred_element_type=jnp.float32)

    @pl.when(j == 0)
    def _():
        acc_ref[...] = jnp.zeros_like(acc_ref)

    acc_ref[...] += o

    @pl.when(j == nj - 1)
    def _():
        sg = jax.nn.sigmoid(
            jax.lax.dot_general(x, gw_ref[...], (((1,), (1,)), ((), ())),
                                preferred_element_type=jnp.float32))
        out_ref[...] = eo_ref[...] + sg * acc_ref[...]


@jax.jit
def kernel(hidden_states, router_weight, merged_gate_up_proj,
           merged_down_proj, shared_gate_up_w, shared_down_w, shared_gate_w):
    x = hidden_states

    c = pl.pallas_call(
        _router_body,
        out_shape=jax.ShapeDtypeStruct((N, E), jnp.float32),
    )(x, router_weight)

    nt = N // TOK_TILE
    expert_out = pl.pallas_call(
        _expert_body,
        grid=(nt, E),
        in_specs=[
            pl.BlockSpec((TOK_TILE, D), lambda t, e: (t, 0)),
            pl.BlockSpec((TOK_TILE, E), lambda t, e: (t, 0)),
            pl.BlockSpec((1, D, 2 * F), lambda t, e: (e, 0, 0)),
            pl.BlockSpec((1, F, D), lambda t, e: (e, 0, 0)),
        ],
        out_specs=pl.BlockSpec((TOK_TILE, D), lambda t, e: (t, 0)),
        out_shape=jax.ShapeDtypeStruct((N, D), jnp.float32),
        compiler_params=pltpu.CompilerParams(
            dimension_semantics=("parallel", "arbitrary")),
    )(x, c, merged_gate_up_proj, merged_down_proj)

    wg = shared_gate_up_w[:FS]
    wu = shared_gate_up_w[FS:]
    nj = FS // FS_CHUNK
    out = pl.pallas_call(
        _shared_body,
        grid=(nt, nj),
        in_specs=[
            pl.BlockSpec((TOK_TILE, D), lambda t, j: (t, 0)),
            pl.BlockSpec((FS_CHUNK, D), lambda t, j: (j, 0)),
            pl.BlockSpec((FS_CHUNK, D), lambda t, j: (j, 0)),
            pl.BlockSpec((D, FS_CHUNK), lambda t, j: (0, j)),
            pl.BlockSpec((TOK_TILE, D), lambda t, j: (t, 0)),
            pl.BlockSpec((1, D), lambda t, j: (0, 0)),
        ],
        out_specs=pl.BlockSpec((TOK_TILE, D), lambda t, j: (t, 0)),
        out_shape=jax.ShapeDtypeStruct((N, D), jnp.float32),
        scratch_shapes=[pltpu.VMEM((TOK_TILE, D), jnp.float32)],
        compiler_params=pltpu.CompilerParams(
            dimension_semantics=("parallel", "arbitrary")),
    )(x, wg, wu, shared_down_w, expert_out, shared_gate_w)

    return out
